# baseline (speedup 1.0000x reference)
import functools

import jax
import jax.numpy as jnp
from jax import lax
from jax.experimental import pallas as pl
from jax.experimental.pallas import tpu as pltpu
from jax.experimental.pallas import tpu_sc as plsc

_REP = 128
_LANES = 16
_BB = 1024
_L_TC = 96


def _sc_part(table, B, D):
    L = table.shape[0]
    info = plsc.get_sparse_core_info()
    n_cores, n_sub = info.num_cores, info.num_subcores
    n_workers = n_cores * n_sub
    rows_per_w = L // n_workers
    n_chunks = B // _REP

    mesh = plsc.VectorSubcoreMesh(core_axis_name="c", subcore_axis_name="s")

    @functools.partial(
        pl.kernel,
        out_type=jax.ShapeDtypeStruct((L, B, D), table.dtype),
        mesh=mesh,
        scratch_types=[
            pltpu.VMEM((1, D), table.dtype),
            pltpu.VMEM((_REP, D), table.dtype),
            pltpu.SemaphoreType.DMA,
        ],
    )
    def sc_embed(table_hbm, out_hbm, row_v, rep_v, sem):
        wid = lax.axis_index("s") * n_cores + lax.axis_index("c")

        def do_row(j, carry):
            l = wid * rows_per_w + j
            pltpu.sync_copy(table_hbm.at[pl.ds(l, 1)], row_v)

            def fill(i, c2):
                for v in range(D // _LANES):
                    rep_v[i, pl.ds(v * _LANES, _LANES)] = row_v[
                        0, pl.ds(v * _LANES, _LANES)
                    ]
                return c2

            lax.fori_loop(0, _REP, fill, 0)

            def fire(i, c2):
                pltpu.make_async_copy(
                    rep_v, out_hbm.at[l, pl.ds(i * _REP, _REP)], sem
                ).start()
                return c2

            lax.fori_loop(0, n_chunks, fire, 0)

            def drain(i, c2):
                pltpu.make_async_copy(
                    rep_v, out_hbm.at[l, pl.ds(i * _REP, _REP)], sem
                ).wait()
                return c2

            lax.fori_loop(0, n_chunks, drain, 0)
            return carry

        lax.fori_loop(0, rows_per_w, do_row, 0)

    return sc_embed(table)


def _tc_copies(tile_ref, o_ref, sem_ref, l, p, nc, bb):
    return [
        pltpu.make_async_copy(
            tile_ref.at[p],
            o_ref.at[l, pl.ds(i * bb, bb), :],
            sem_ref.at[p, i],
        )
        for i in range(nc)
    ]


def _tc_fanout_kernel(w_ref, o_ref, tile_ref, sem_ref):
    nbuf, bb, d = tile_ref.shape
    nc = sem_ref.shape[1]
    n_l = pl.num_programs(0)
    l = pl.program_id(0)
    p = jax.lax.rem(l, 2)

    @pl.when(l >= 2)
    def _():
        for c in _tc_copies(tile_ref, o_ref, sem_ref, l - 2, p, nc, bb):
            c.wait()

    tile_ref[p] = jnp.broadcast_to(w_ref[0], (bb, d))
    for c in _tc_copies(tile_ref, o_ref, sem_ref, l, p, nc, bb):
        c.start()

    @pl.when(l == n_l - 1)
    def _():
        for c in _tc_copies(tile_ref, o_ref, sem_ref, l - 1, 1 - p, nc, bb):
            c.wait()
        for c in _tc_copies(tile_ref, o_ref, sem_ref, l, p, nc, bb):
            c.wait()


def _tc_part(table, B, D):
    L = table.shape[0]
    nc = B // _BB
    w3 = table.reshape(L, 1, D)
    return pl.pallas_call(
        _tc_fanout_kernel,
        grid=(L,),
        in_specs=[pl.BlockSpec((1, 1, D), lambda l: (l, 0, 0))],
        out_specs=pl.BlockSpec(memory_space=pltpu.MemorySpace.HBM),
        out_shape=jax.ShapeDtypeStruct((L, B, D), table.dtype),
        scratch_shapes=[
            pltpu.VMEM((2, _BB, D), table.dtype),
            pltpu.SemaphoreType.DMA((2, nc)),
        ],
    )(w3)


def kernel(mask, embed_weight):
    B, L = mask.shape
    D = embed_weight.shape[-1]
    sc_out = _sc_part(embed_weight[_L_TC:], B, D)
    tc_out = _tc_part(embed_weight[:_L_TC], B, D)
    return jnp.concatenate([tc_out, sc_out], axis=0)


# speedup vs baseline: 2.1894x; 2.1894x over previous
import functools

import jax
import jax.numpy as jnp
from jax import lax
from jax.experimental import pallas as pl
from jax.experimental.pallas import tpu as pltpu
from jax.experimental.pallas import tpu_sc as plsc

_REP = 256
_LANES = 16


def kernel(mask, embed_weight):
    B, L = mask.shape
    D = embed_weight.shape[-1]
    info = plsc.get_sparse_core_info()
    n_cores, n_sub = info.num_cores, info.num_subcores
    n_workers = n_cores * n_sub
    rows_per_w = L // n_workers
    n_chunks = B // _REP

    mesh = plsc.VectorSubcoreMesh(core_axis_name="c", subcore_axis_name="s")

    @functools.partial(
        pl.kernel,
        out_type=jax.ShapeDtypeStruct((L, B, D), embed_weight.dtype),
        mesh=mesh,
        scratch_types=[
            pltpu.VMEM((1, D), embed_weight.dtype),
            pltpu.VMEM((_REP, D), embed_weight.dtype),
            pltpu.SemaphoreType.DMA,
        ],
    )
    def sc_embed(table_hbm, out_hbm, row_v, rep_v, sem):
        wid = lax.axis_index("s") * n_cores + lax.axis_index("c")

        def do_row(j, carry):
            l = wid * rows_per_w + j
            pltpu.sync_copy(table_hbm.at[pl.ds(l, 1)], row_v)

            def fill(i, c2):
                for v in range(D // _LANES):
                    rep_v[i, pl.ds(v * _LANES, _LANES)] = row_v[
                        0, pl.ds(v * _LANES, _LANES)
                    ]
                return c2

            lax.fori_loop(0, _REP, fill, 0)

            def fire(i, c2):
                pltpu.make_async_copy(
                    rep_v, out_hbm.at[l, pl.ds(i * _REP, _REP)], sem
                ).start()
                return c2

            lax.fori_loop(0, n_chunks, fire, 0)

            def drain(i, c2):
                pltpu.make_async_copy(
                    rep_v, out_hbm.at[l, pl.ds(i * _REP, _REP)], sem
                ).wait()
                return c2

            lax.fori_loop(0, n_chunks, drain, 0)
            return carry

        lax.fori_loop(0, rows_per_w, do_row, 0)

    return sc_embed(embed_weight)


# speedup vs baseline: 2.7330x; 1.2483x over previous
import functools

import jax
import jax.numpy as jnp
from jax import lax
from jax.experimental import pallas as pl
from jax.experimental.pallas import tpu as pltpu
from jax.experimental.pallas import tpu_sc as plsc

_REP = 128
_LANES = 16


def kernel(mask, embed_weight):
    B, L = mask.shape
    D = embed_weight.shape[-1]
    info = plsc.get_sparse_core_info()
    n_cores, n_sub = info.num_cores, info.num_subcores
    n_workers = n_cores * n_sub
    rows_per_w = L // n_workers
    n_chunks = B // _REP

    mesh = plsc.VectorSubcoreMesh(core_axis_name="c", subcore_axis_name="s")

    @functools.partial(
        pl.kernel,
        out_type=jax.ShapeDtypeStruct((L, B, D), embed_weight.dtype),
        mesh=mesh,
        scratch_types=[
            pltpu.VMEM((1, D), embed_weight.dtype),
            pltpu.VMEM((2, _REP, D), embed_weight.dtype),
            pltpu.SemaphoreType.DMA((2,)),
        ],
    )
    def sc_embed(table_hbm, out_hbm, row_v, rep_v, sem):
        wid = lax.axis_index("s") * n_cores + lax.axis_index("c")
        row0 = wid * rows_per_w

        def fill(i, p):
            for v in range(D // _LANES):
                rep_v[p, i, pl.ds(v * _LANES, _LANES)] = row_v[
                    0, pl.ds(v * _LANES, _LANES)
                ]
            return p

        def fire(i, args):
            l, p = args
            pltpu.make_async_copy(
                rep_v.at[p], out_hbm.at[l, pl.ds(i * _REP, _REP)], sem.at[p]
            ).start()
            return args

        def drain(i, args):
            l, p = args
            pltpu.make_async_copy(
                rep_v.at[p], out_hbm.at[l, pl.ds(i * _REP, _REP)], sem.at[p]
            ).wait()
            return args

        for j in range(rows_per_w):
            p = j % 2
            if j >= 2:
                lax.fori_loop(0, n_chunks, drain, (row0 + j - 2, p))
            pltpu.sync_copy(table_hbm.at[pl.ds(row0 + j, 1)], row_v)
            lax.fori_loop(0, _REP, fill, p)
            lax.fori_loop(0, n_chunks, fire, (row0 + j, p))

        for j in range(max(rows_per_w - 2, 0), rows_per_w):
            lax.fori_loop(0, n_chunks, drain, (row0 + j, j % 2))

    return sc_embed(embed_weight)


# speedup vs baseline: 2.7445x; 1.0042x over previous
import functools

import jax
import jax.numpy as jnp
from jax import lax
from jax.experimental import pallas as pl
from jax.experimental.pallas import tpu as pltpu
from jax.experimental.pallas import tpu_sc as plsc

_REP = 128
_CHUNK = 64
_LANES = 16


def kernel(mask, embed_weight):
    B, L = mask.shape
    D = embed_weight.shape[-1]
    info = plsc.get_sparse_core_info()
    n_cores, n_sub = info.num_cores, info.num_subcores
    n_workers = n_cores * n_sub
    rows_per_w = L // n_workers
    n_chunks = B // _CHUNK
    reps_per_buf = _REP // _CHUNK

    mesh = plsc.VectorSubcoreMesh(core_axis_name="c", subcore_axis_name="s")

    @functools.partial(
        pl.kernel,
        out_type=jax.ShapeDtypeStruct((L, B, D), embed_weight.dtype),
        mesh=mesh,
        scratch_types=[
            pltpu.VMEM((1, D), embed_weight.dtype),
            pltpu.VMEM((2, _REP, D), embed_weight.dtype),
            pltpu.SemaphoreType.DMA((2,)),
        ],
    )
    def sc_embed(table_hbm, out_hbm, row_v, rep_v, sem):
        wid = lax.axis_index("s") * n_cores + lax.axis_index("c")
        row0 = wid * rows_per_w

        def fill(i, p):
            for v in range(D // _LANES):
                rep_v[p, i, pl.ds(v * _LANES, _LANES)] = row_v[
                    0, pl.ds(v * _LANES, _LANES)
                ]
            return p

        def fire(i, args):
            l, p = args
            src = rep_v.at[p, pl.ds(lax.rem(i, reps_per_buf) * _CHUNK, _CHUNK)]
            pltpu.make_async_copy(
                src, out_hbm.at[l, pl.ds(i * _CHUNK, _CHUNK)], sem.at[p]
            ).start()
            return args

        def drain(i, args):
            l, p = args
            src = rep_v.at[p, pl.ds(lax.rem(i, reps_per_buf) * _CHUNK, _CHUNK)]
            pltpu.make_async_copy(
                src, out_hbm.at[l, pl.ds(i * _CHUNK, _CHUNK)], sem.at[p]
            ).wait()
            return args

        for j in range(rows_per_w):
            p = j % 2
            if j >= 2:
                lax.fori_loop(0, n_chunks, drain, (row0 + j - 2, p))
            pltpu.sync_copy(table_hbm.at[pl.ds(row0 + j, 1)], row_v)
            lax.fori_loop(0, _REP, fill, p)
            lax.fori_loop(0, n_chunks, fire, (row0 + j, p))

        for j in range(max(rows_per_w - 2, 0), rows_per_w):
            lax.fori_loop(0, n_chunks, drain, (row0 + j, j % 2))

    return sc_embed(embed_weight)


# speedup vs baseline: 2.7932x; 1.0177x over previous
import functools

import jax
import jax.numpy as jnp
from jax import lax
from jax.experimental import pallas as pl
from jax.experimental.pallas import tpu as pltpu
from jax.experimental.pallas import tpu_sc as plsc

_REP = 128
_CHUNK = 64
_LANES = 16


def kernel(mask, embed_weight):
    B, L = mask.shape
    D = embed_weight.shape[-1]
    info = plsc.get_sparse_core_info()
    n_cores, n_sub = info.num_cores, info.num_subcores
    n_workers = n_cores * n_sub
    rows_per_w = L // n_workers
    n_chunks = B // _CHUNK
    reps_per_buf = _REP // _CHUNK

    mesh = plsc.VectorSubcoreMesh(core_axis_name="c", subcore_axis_name="s")

    @functools.partial(
        pl.kernel,
        out_type=jax.ShapeDtypeStruct((L, B, D), embed_weight.dtype),
        mesh=mesh,
        scratch_types=[
            pltpu.VMEM((L // n_workers, D), embed_weight.dtype),
            pltpu.VMEM((2, _REP, D), embed_weight.dtype),
            pltpu.SemaphoreType.DMA((2,)),
        ],
    )
    def sc_embed(table_hbm, out_hbm, rows_v, rep_v, sem):
        wid = lax.axis_index("s") * n_cores + lax.axis_index("c")
        row0 = wid * rows_per_w
        for j in range(rows_per_w):
            pltpu.sync_copy(
                table_hbm.at[pl.ds(row0 + j, 1)], rows_v.at[pl.ds(j, 1)]
            )

        def fire(i, args):
            l, p = args
            src = rep_v.at[p, pl.ds(lax.rem(i, reps_per_buf) * _CHUNK, _CHUNK)]
            pltpu.make_async_copy(
                src, out_hbm.at[l, pl.ds(i * _CHUNK, _CHUNK)], sem.at[p]
            ).start()
            return args

        def drain(i, args):
            l, p = args
            src = rep_v.at[p, pl.ds(lax.rem(i, reps_per_buf) * _CHUNK, _CHUNK)]
            pltpu.make_async_copy(
                src, out_hbm.at[l, pl.ds(i * _CHUNK, _CHUNK)], sem.at[p]
            ).wait()
            return args

        for j in range(rows_per_w):
            p = j % 2
            if j >= 2:
                lax.fori_loop(0, n_chunks, drain, (row0 + j - 2, p))
            vecs = [
                rows_v[j, pl.ds(v * _LANES, _LANES)] for v in range(D // _LANES)
            ]

            def fill(i, p2, vecs=vecs):
                for v in range(D // _LANES):
                    rep_v[p2, i, pl.ds(v * _LANES, _LANES)] = vecs[v]
                return p2

            lax.fori_loop(0, _REP, fill, p)
            lax.fori_loop(0, n_chunks, fire, (row0 + j, p))

        for j in range(max(rows_per_w - 2, 0), rows_per_w):
            lax.fori_loop(0, n_chunks, drain, (row0 + j, j % 2))

    return sc_embed(embed_weight)
